# baseline (speedup 1.0000x reference)
import functools

import jax
import jax.numpy as jnp
from jax.experimental import pallas as pl
from jax.experimental.pallas import tpu as pltpu

B = 32768
D = 4096
E = 64
K = 8
TB = 256


def _router_body(f_ref, w_ref, b_ref, lora_ref, o_ref):
    f = f_ref[...]
    w = w_ref[...]
    scores = jax.lax.dot_general(
        f, w, (((1,), (1,)), ((), ())), preferred_element_type=jnp.float32
    )
    scores = scores + b_ref[...]

    work = scores
    m = None
    for _ in range(K):
        m = jnp.max(work, axis=1, keepdims=True)
        work = jnp.where(work >= m, -jnp.inf, work)
    mask = scores >= m

    mx = jnp.max(jnp.where(mask, scores, -jnp.inf), axis=1, keepdims=True)
    ex = jnp.where(mask, jnp.exp(scores - mx), 0.0)
    weights = ex / jnp.sum(ex, axis=1, keepdims=True)

    combined = jax.lax.dot_general(
        weights, lora_ref[...], (((1,), (0,)), ((), ())),
        preferred_element_type=jnp.float32,
    )
    o_ref[...] = f + combined


@jax.jit
def kernel(features, W_attn, b_attn, lora_ranks):
    b2 = b_attn.reshape(1, E)
    grid = (B // TB,)
    return pl.pallas_call(
        _router_body,
        grid=grid,
        in_specs=[
            pl.BlockSpec((TB, D), lambda i: (i, 0)),
            pl.BlockSpec((E, D), lambda i: (0, 0)),
            pl.BlockSpec((1, E), lambda i: (0, 0)),
            pl.BlockSpec((E, D), lambda i: (0, 0)),
        ],
        out_specs=pl.BlockSpec((TB, D), lambda i: (i, 0)),
        out_shape=jax.ShapeDtypeStruct((B, D), jnp.float32),
        compiler_params=pltpu.CompilerParams(
            dimension_semantics=("arbitrary",),
        ),
    )(features, W_attn, b2, lora_ranks)


# speedup vs baseline: 1.3269x; 1.3269x over previous
import jax
import jax.numpy as jnp
from jax.experimental import pallas as pl
from jax.experimental.pallas import tpu as pltpu

B = 32768
D = 4096
E = 64
K = 8
TB = 512


def _router_body(f_ref, w_ref, b_ref, lora_ref, o_ref):
    f = f_ref[...]
    st = jax.lax.dot_general(
        w_ref[...], f, (((1,), (1,)), ((), ())), preferred_element_type=jnp.float32
    ) + b_ref[...]

    work = st
    m = None
    for _ in range(K):
        m = jnp.max(work, axis=0, keepdims=True)
        work = jnp.where(work >= m, -jnp.inf, work)
    mask = st >= m

    mx = jnp.max(jnp.where(mask, st, -jnp.inf), axis=0, keepdims=True)
    ex = jnp.where(mask, jnp.exp(st - mx), 0.0)
    wts = ex / jnp.sum(ex, axis=0, keepdims=True)

    combined = jax.lax.dot_general(
        wts, lora_ref[...], (((0,), (0,)), ((), ())),
        preferred_element_type=jnp.float32,
    )
    o_ref[...] = f + combined


@jax.jit
def kernel(features, W_attn, b_attn, lora_ranks):
    b2 = b_attn.reshape(E, 1)
    grid = (B // TB,)
    return pl.pallas_call(
        _router_body,
        grid=grid,
        in_specs=[
            pl.BlockSpec((TB, D), lambda i: (i, 0)),
            pl.BlockSpec((E, D), lambda i: (0, 0)),
            pl.BlockSpec((E, 1), lambda i: (0, 0)),
            pl.BlockSpec((E, D), lambda i: (0, 0)),
        ],
        out_specs=pl.BlockSpec((TB, D), lambda i: (i, 0)),
        out_shape=jax.ShapeDtypeStruct((B, D), jnp.float32),
        compiler_params=pltpu.CompilerParams(
            dimension_semantics=("arbitrary",),
        ),
    )(features, W_attn, b2, lora_ranks)
